# baseline (speedup 1.0000x reference)
import jax
import jax.numpy as jnp
from jax.experimental import pallas as pl
from jax.experimental.pallas import tpu as pltpu

C = 32
_VMEM = 100 * 1024 * 1024
_PARAMS1 = pltpu.CompilerParams(dimension_semantics=("arbitrary",),
                                vmem_limit_bytes=_VMEM)
_PARAMS2 = pltpu.CompilerParams(dimension_semantics=("parallel", "arbitrary"),
                                vmem_limit_bytes=_VMEM)


def _bf(v):
    return v.astype(jnp.bfloat16)


def _cheb_chain(a, xs, w_r, w_z1, w_z2, y_prev, bm=256, nc=None):
    x = xs[0] if len(xs) == 1 else jnp.concatenate(xs, axis=1)
    m = a.shape[0]
    n = x.shape[1]
    ni = m // bm
    if nc is None:
        nc = ni
    nsteps = ni + (ni - nc)

    def body(*refs):
        a_ref = refs[0]
        x_ref = refs[1]
        pos = 2
        wr_ref = refs[pos] if w_r is not None else None
        pos += 1 if w_r is not None else 0
        wz1_ref, wz2_ref = refs[pos], refs[pos + 1]
        pos += 2
        yp_ref = refs[pos] if y_prev is not None else None
        o_ref = refs[-4]
        lbf_ref, rbf_ref, z1bf_ref = refs[-3:]

        i = pl.program_id(0)

        @pl.when(i == 0)
        def _build_r():
            rbf_ref[...] = _bf(x_ref[...])

        def _emit_row_block(sl, z2b):
            yb = jnp.dot(_bf(z2b), _bf(wz2_ref[...]),
                         preferred_element_type=jnp.float32)
            yb += jnp.dot(z1bf_ref[sl, :], _bf(wz1_ref[...]),
                          preferred_element_type=jnp.float32)
            if wr_ref is not None:
                yb += jnp.dot(rbf_ref[sl, :], _bf(wr_ref[...]),
                              preferred_element_type=jnp.float32)
            if yp_ref is not None:
                yb += yp_ref[sl, :]
            o_ref[sl, :] = yb

        @pl.when(i < ni)
        def _pass1():
            ab = _bf(a_ref[...])

            @pl.when(i < nc)
            def _cache():
                lbf_ref[pl.ds(i * bm, bm), :] = ab

            z1bf_ref[pl.ds(i * bm, bm), :] = _bf(jnp.dot(
                ab, rbf_ref[...], preferred_element_type=jnp.float32))

        @pl.when(i == ni - 1)
        def _emit_cached():
            for ib in range(nc):
                sl = pl.ds(ib * bm, bm)
                z2b = jnp.dot(lbf_ref[sl, :], z1bf_ref[...],
                              preferred_element_type=jnp.float32)
                _emit_row_block(sl, z2b)

        @pl.when(i >= ni)
        def _emit_streamed():
            sl = pl.ds((i - ni + nc) * bm, bm)
            z2b = jnp.dot(_bf(a_ref[...]), z1bf_ref[...],
                          preferred_element_type=jnp.float32)
            _emit_row_block(sl, z2b)

    def _a_index(i):
        return (jnp.where(i < ni, i, i - ni + nc), 0)

    in_specs = [pl.BlockSpec((bm, m), _a_index),
                pl.BlockSpec(x.shape, lambda i: (0, 0))]
    args = [a, x]
    if w_r is not None:
        in_specs.append(pl.BlockSpec(w_r.shape, lambda i: (0, 0)))
        args.append(w_r)
    in_specs.append(pl.BlockSpec(w_z1.shape, lambda i: (0, 0)))
    args.append(w_z1)
    in_specs.append(pl.BlockSpec(w_z2.shape, lambda i: (0, 0)))
    args.append(w_z2)
    if y_prev is not None:
        in_specs.append(pl.BlockSpec((m, C), lambda i: (0, 0)))
        args.append(y_prev)
    return pl.pallas_call(
        body, grid=(nsteps,),
        in_specs=in_specs,
        out_specs=pl.BlockSpec((m, C), lambda i: (0, 0)),
        out_shape=jax.ShapeDtypeStruct((m, C), jnp.float32),
        scratch_shapes=[pltpu.VMEM((nc * bm, m), jnp.bfloat16),
                        pltpu.VMEM((m, n), jnp.bfloat16),
                        pltpu.VMEM((m, n), jnp.bfloat16)],
        compiler_params=_PARAMS1,
    )(*args)


def _lap_pass(a, xs, bm=1024, bk=2048):
    m, k = a.shape
    bm = min(bm, m)
    bk = min(bk, k)
    n = sum(x.shape[1] for x in xs)

    def body(a_ref, *rest):
        x_refs, o_ref = rest[:-1], rest[-1]

        @pl.when(pl.program_id(1) == 0)
        def _init():
            o_ref[...] = jnp.zeros_like(o_ref)

        j = pl.program_id(1)
        xb = jnp.concatenate(
            [_bf(x[pl.ds(j * bk, bk), :]) for x in x_refs], axis=1)
        o_ref[...] += jnp.dot(_bf(a_ref[...]), xb,
                              preferred_element_type=jnp.float32)

    in_specs = [pl.BlockSpec((bm, bk), lambda i, j: (i, j))] + [
        pl.BlockSpec(x.shape, lambda i, j: (0, 0)) for x in xs]
    return pl.pallas_call(
        body, grid=(m // bm, k // bk),
        in_specs=in_specs,
        out_specs=pl.BlockSpec((bm, n), lambda i, j: (i, 0)),
        out_shape=jax.ShapeDtypeStruct((m, n), jnp.float32),
        compiler_params=_PARAMS2,
    )(a, *xs)


def _lap_pass2_combine(a, x2, statics, w_acc, w_static, y_prev,
                       bm=1024, bk=2048):
    m, k = a.shape
    bm = min(bm, m)
    bk = min(bk, k)
    nacc = x2.shape[1]

    def body(*refs):
        a_ref = refs[0]
        x2_ref = refs[1]
        pos = 2
        s_refs = refs[pos:pos + len(statics)]
        pos += len(statics)
        wa_ref = refs[pos]
        pos += 1
        ws_ref = refs[pos] if w_static is not None else None
        pos += 1 if w_static is not None else 0
        yp_ref = refs[pos] if y_prev is not None else None
        o_ref, acc_ref = refs[-2], refs[-1]

        j = pl.program_id(1)
        d = jnp.dot(_bf(a_ref[...]), _bf(x2_ref[pl.ds(j * bk, bk), :]),
                    preferred_element_type=jnp.float32)

        @pl.when(j == 0)
        def _init():
            acc_ref[...] = d

        @pl.when(j > 0)
        def _acc():
            acc_ref[...] += d

        @pl.when(j == pl.num_programs(1) - 1)
        def _emit():
            y = jnp.dot(_bf(acc_ref[...]), _bf(wa_ref[...]),
                        preferred_element_type=jnp.float32)
            if ws_ref is not None:
                xs = jnp.concatenate([_bf(s[...]) for s in s_refs], axis=1)
                y += jnp.dot(xs, _bf(ws_ref[...]),
                             preferred_element_type=jnp.float32)
            if yp_ref is not None:
                y += yp_ref[...]
            o_ref[...] = y

    in_specs = [pl.BlockSpec((bm, bk), lambda i, j: (i, j)),
                pl.BlockSpec(x2.shape, lambda i, j: (0, 0))]
    args = [a, x2]
    for s in statics:
        in_specs.append(pl.BlockSpec((bm, s.shape[1]), lambda i, j: (i, 0)))
        args.append(s)
    in_specs.append(pl.BlockSpec(w_acc.shape, lambda i, j: (0, 0)))
    args.append(w_acc)
    if w_static is not None:
        in_specs.append(pl.BlockSpec(w_static.shape, lambda i, j: (0, 0)))
        args.append(w_static)
    if y_prev is not None:
        in_specs.append(pl.BlockSpec((bm, C), lambda i, j: (i, 0)))
        args.append(y_prev)
    return pl.pallas_call(
        body, grid=(m // bm, k // bk),
        in_specs=in_specs,
        out_specs=pl.BlockSpec((bm, C), lambda i, j: (i, 0)),
        out_shape=jax.ShapeDtypeStruct((m, C), jnp.float32),
        scratch_shapes=[pltpu.VMEM((bm, nacc), jnp.float32)],
        compiler_params=_PARAMS2,
    )(*args)


def _dual(b, xr, xc, bk):
    m, k = b.shape

    def body(b_ref, xr_ref, xc_ref, u_ref, l_ref):
        @pl.when(pl.program_id(0) == 0)
        def _init():
            u_ref[...] = jnp.zeros_like(u_ref)

        bb = _bf(b_ref[...])
        u_ref[...] += jnp.dot(bb, _bf(xr_ref[...]),
                              preferred_element_type=jnp.float32)
        l_ref[...] = jax.lax.dot_general(
            bb, _bf(xc_ref[...]), (((0,), (0,)), ((), ())),
            preferred_element_type=jnp.float32)

    return pl.pallas_call(
        body, grid=(k // bk,),
        in_specs=[pl.BlockSpec((m, bk), lambda j: (0, j)),
                  pl.BlockSpec((bk, C), lambda j: (j, 0)),
                  pl.BlockSpec((m, C), lambda j: (0, 0))],
        out_specs=[pl.BlockSpec((m, C), lambda j: (0, 0)),
                   pl.BlockSpec((bk, C), lambda j: (j, 0))],
        out_shape=[jax.ShapeDtypeStruct((m, C), jnp.float32),
                   jax.ShapeDtypeStruct((k, C), jnp.float32)],
        compiler_params=_PARAMS1,
    )(b, xr, xc)


def _rank0_chain(lap, x0, u1, w0):
    m = lap.shape[0]

    def body(l_ref, x0_ref, u1_ref, w_ref, o_ref):
        lb = _bf(l_ref[...])
        r0 = jnp.concatenate([_bf(x0_ref[...]), _bf(u1_ref[...])], axis=1)
        z1 = jnp.dot(lb, r0, preferred_element_type=jnp.float32)
        z2 = jnp.dot(lb, _bf(z1), preferred_element_type=jnp.float32)
        xall = jnp.concatenate([r0, _bf(z1), _bf(z2)], axis=1)
        o_ref[...] = jnp.dot(xall, _bf(w_ref[...]),
                             preferred_element_type=jnp.float32)

    return pl.pallas_call(
        body,
        in_specs=[pl.BlockSpec(lap.shape, lambda: (0, 0)),
                  pl.BlockSpec((m, C), lambda: (0, 0)),
                  pl.BlockSpec((m, C), lambda: (0, 0)),
                  pl.BlockSpec(w0.shape, lambda: (0, 0))],
        out_specs=pl.BlockSpec((m, C), lambda: (0, 0)),
        out_shape=jax.ShapeDtypeStruct((m, C), jnp.float32),
        compiler_params=pltpu.CompilerParams(vmem_limit_bytes=_VMEM),
    )(lap, x0, u1, w0)


def kernel(x_0, x_1, x_2, laplacian_0, laplacian_down_1, laplacian_up_1,
           laplacian_down_2, laplacian_up_2, b1, b2,
           weight_0, weight_1, weight_2):
    u1, l1 = _dual(b1, x_1, x_0, bk=1024)
    u2, l2 = _dual(b2, x_2, x_1, bk=256)

    w0 = weight_0.transpose(2, 0, 1)
    w0 = w0[jnp.array([0, 3, 1, 4, 2, 5])].reshape(6 * C, C)
    y_0 = _rank0_chain(laplacian_0, x_0, u1, w0)

    w1 = weight_1.transpose(2, 0, 1)
    w1 = w1.at[6].add(w1[8]).at[7].add(w1[9])
    zd1a = _lap_pass(laplacian_down_1, [x_1, l1, u2])
    zu1a = _lap_pass(laplacian_up_1, [l1, u2])
    ws1 = w1[jnp.array([0, 5, 10, 6, 1, 11, 3, 13])].reshape(8 * C, C)
    wd1 = w1[jnp.array([7, 2, 12])].reshape(3 * C, C)
    wu1 = w1[jnp.array([4, 14])].reshape(2 * C, C)
    y1p = _lap_pass2_combine(laplacian_down_1, zd1a,
                             [l1, x_1, u2, zd1a, zu1a], wd1, ws1, None)
    y_1 = _lap_pass2_combine(laplacian_up_1, zu1a, [], wu1, None, y1p)

    w2 = weight_2.transpose(2, 0, 1)
    w2 = w2.at[1].add(w2[3]).at[2].add(w2[4])
    wr2 = w2[jnp.array([5, 0])].reshape(2 * C, C)
    wz1d2 = w2[jnp.array([6, 1])].reshape(2 * C, C)
    wz2d2 = w2[jnp.array([7, 2])].reshape(2 * C, C)
    wz1u2 = w2[jnp.array([8])].reshape(C, C)
    wz2u2 = w2[jnp.array([9])].reshape(C, C)
    y2p = _cheb_chain(laplacian_down_2, [x_2, l2], wr2, wz1d2, wz2d2, None, bm=512)
    y_2 = _cheb_chain(laplacian_up_2, [x_2], None, wz1u2, wz2u2, y2p, bm=512)

    return (y_0, y_1, y_2)


# speedup vs baseline: 1.0264x; 1.0264x over previous
import jax
import jax.numpy as jnp
from jax.experimental import pallas as pl
from jax.experimental.pallas import tpu as pltpu

C = 32
_VMEM = 100 * 1024 * 1024
_PARAMS1 = pltpu.CompilerParams(dimension_semantics=("arbitrary",),
                                vmem_limit_bytes=_VMEM)
_PARAMS2 = pltpu.CompilerParams(dimension_semantics=("parallel", "arbitrary"),
                                vmem_limit_bytes=_VMEM)


def _bf(v):
    return v.astype(jnp.bfloat16)


def _cheb_chain(a, xs, w_r, w_z1, w_z2, y_prev, bm=256, nc=None):
    m = a.shape[0]
    n = sum(x.shape[1] for x in xs)
    ni = m // bm
    if nc is None:
        nc = ni
    nsteps = ni + (ni - nc)

    def body(*refs):
        a_ref = refs[0]
        x_refs = refs[1:1 + len(xs)]
        pos = 1 + len(xs)
        wr_ref = refs[pos] if w_r is not None else None
        pos += 1 if w_r is not None else 0
        wz1_ref, wz2_ref = refs[pos], refs[pos + 1]
        pos += 2
        yp_ref = refs[pos] if y_prev is not None else None
        o_ref = refs[-4]
        lbf_ref, rbf_ref, z1bf_ref = refs[-3:]

        i = pl.program_id(0)

        @pl.when(i == 0)
        def _build_r():
            rbf_ref[...] = jnp.concatenate(
                [_bf(x[...]) for x in x_refs], axis=1)

        def _emit_row_block(sl, z2b):
            yb = jnp.dot(_bf(z2b), _bf(wz2_ref[...]),
                         preferred_element_type=jnp.float32)
            yb += jnp.dot(z1bf_ref[sl, :], _bf(wz1_ref[...]),
                          preferred_element_type=jnp.float32)
            if wr_ref is not None:
                yb += jnp.dot(rbf_ref[sl, :], _bf(wr_ref[...]),
                              preferred_element_type=jnp.float32)
            if yp_ref is not None:
                yb += yp_ref[sl, :]
            o_ref[sl, :] = yb

        @pl.when(i < ni)
        def _pass1():
            ab = _bf(a_ref[...])

            @pl.when(i < nc)
            def _cache():
                lbf_ref[pl.ds(i * bm, bm), :] = ab

            z1bf_ref[pl.ds(i * bm, bm), :] = _bf(jnp.dot(
                ab, rbf_ref[...], preferred_element_type=jnp.float32))

        @pl.when(i == ni - 1)
        def _emit_cached():
            for ib in range(nc):
                sl = pl.ds(ib * bm, bm)
                z2b = jnp.dot(lbf_ref[sl, :], z1bf_ref[...],
                              preferred_element_type=jnp.float32)
                _emit_row_block(sl, z2b)

        @pl.when(i >= ni)
        def _emit_streamed():
            sl = pl.ds((i - ni + nc) * bm, bm)
            z2b = jnp.dot(_bf(a_ref[...]), z1bf_ref[...],
                          preferred_element_type=jnp.float32)
            _emit_row_block(sl, z2b)

    def _a_index(i):
        return (jnp.where(i < ni, i, i - ni + nc), 0)

    in_specs = [pl.BlockSpec((bm, m), _a_index)]
    args = [a]
    for x in xs:
        in_specs.append(pl.BlockSpec(x.shape, lambda i: (0, 0)))
        args.append(x)
    if w_r is not None:
        in_specs.append(pl.BlockSpec(w_r.shape, lambda i: (0, 0)))
        args.append(w_r)
    in_specs.append(pl.BlockSpec(w_z1.shape, lambda i: (0, 0)))
    args.append(w_z1)
    in_specs.append(pl.BlockSpec(w_z2.shape, lambda i: (0, 0)))
    args.append(w_z2)
    if y_prev is not None:
        in_specs.append(pl.BlockSpec((m, C), lambda i: (0, 0)))
        args.append(y_prev)
    return pl.pallas_call(
        body, grid=(nsteps,),
        in_specs=in_specs,
        out_specs=pl.BlockSpec((m, C), lambda i: (0, 0)),
        out_shape=jax.ShapeDtypeStruct((m, C), jnp.float32),
        scratch_shapes=[pltpu.VMEM((nc * bm, m), jnp.bfloat16),
                        pltpu.VMEM((m, n), jnp.bfloat16),
                        pltpu.VMEM((m, n), jnp.bfloat16)],
        compiler_params=_PARAMS1,
    )(*args)


def _lap_pass(a, xs, bm=1024, bk=2048):
    m, k = a.shape
    bm = min(bm, m)
    bk = min(bk, k)
    n = sum(x.shape[1] for x in xs)

    def body(a_ref, *rest):
        x_refs, o_ref = rest[:-1], rest[-1]

        @pl.when(pl.program_id(1) == 0)
        def _init():
            o_ref[...] = jnp.zeros_like(o_ref)

        j = pl.program_id(1)
        xb = jnp.concatenate(
            [_bf(x[pl.ds(j * bk, bk), :]) for x in x_refs], axis=1)
        o_ref[...] += jnp.dot(_bf(a_ref[...]), xb,
                              preferred_element_type=jnp.float32)

    in_specs = [pl.BlockSpec((bm, bk), lambda i, j: (i, j))] + [
        pl.BlockSpec(x.shape, lambda i, j: (0, 0)) for x in xs]
    return pl.pallas_call(
        body, grid=(m // bm, k // bk),
        in_specs=in_specs,
        out_specs=pl.BlockSpec((bm, n), lambda i, j: (i, 0)),
        out_shape=jax.ShapeDtypeStruct((m, n), jnp.float32),
        compiler_params=_PARAMS2,
    )(a, *xs)


def _lap_pass2_combine(a, x2, statics, w_acc, w_static, y_prev,
                       bm=1024, bk=2048):
    m, k = a.shape
    bm = min(bm, m)
    bk = min(bk, k)
    nacc = x2.shape[1]

    def body(*refs):
        a_ref = refs[0]
        x2_ref = refs[1]
        pos = 2
        s_refs = refs[pos:pos + len(statics)]
        pos += len(statics)
        wa_ref = refs[pos]
        pos += 1
        ws_ref = refs[pos] if w_static is not None else None
        pos += 1 if w_static is not None else 0
        yp_ref = refs[pos] if y_prev is not None else None
        o_ref, acc_ref = refs[-2], refs[-1]

        j = pl.program_id(1)
        d = jnp.dot(_bf(a_ref[...]), _bf(x2_ref[pl.ds(j * bk, bk), :]),
                    preferred_element_type=jnp.float32)

        @pl.when(j == 0)
        def _init():
            acc_ref[...] = d

        @pl.when(j > 0)
        def _acc():
            acc_ref[...] += d

        @pl.when(j == pl.num_programs(1) - 1)
        def _emit():
            y = jnp.dot(_bf(acc_ref[...]), _bf(wa_ref[...]),
                        preferred_element_type=jnp.float32)
            if ws_ref is not None:
                xs = jnp.concatenate([_bf(s[...]) for s in s_refs], axis=1)
                y += jnp.dot(xs, _bf(ws_ref[...]),
                             preferred_element_type=jnp.float32)
            if yp_ref is not None:
                y += yp_ref[...]
            o_ref[...] = y

    in_specs = [pl.BlockSpec((bm, bk), lambda i, j: (i, j)),
                pl.BlockSpec(x2.shape, lambda i, j: (0, 0))]
    args = [a, x2]
    for s in statics:
        in_specs.append(pl.BlockSpec((bm, s.shape[1]), lambda i, j: (i, 0)))
        args.append(s)
    in_specs.append(pl.BlockSpec(w_acc.shape, lambda i, j: (0, 0)))
    args.append(w_acc)
    if w_static is not None:
        in_specs.append(pl.BlockSpec(w_static.shape, lambda i, j: (0, 0)))
        args.append(w_static)
    if y_prev is not None:
        in_specs.append(pl.BlockSpec((bm, C), lambda i, j: (i, 0)))
        args.append(y_prev)
    return pl.pallas_call(
        body, grid=(m // bm, k // bk),
        in_specs=in_specs,
        out_specs=pl.BlockSpec((bm, C), lambda i, j: (i, 0)),
        out_shape=jax.ShapeDtypeStruct((m, C), jnp.float32),
        scratch_shapes=[pltpu.VMEM((bm, nacc), jnp.float32)],
        compiler_params=_PARAMS2,
    )(*args)


def _dual(b, xr, xc, bk):
    m, k = b.shape

    def body(b_ref, xr_ref, xc_ref, u_ref, l_ref):
        @pl.when(pl.program_id(0) == 0)
        def _init():
            u_ref[...] = jnp.zeros_like(u_ref)

        bb = _bf(b_ref[...])
        u_ref[...] += jnp.dot(bb, _bf(xr_ref[...]),
                              preferred_element_type=jnp.float32)
        l_ref[...] = jax.lax.dot_general(
            bb, _bf(xc_ref[...]), (((0,), (0,)), ((), ())),
            preferred_element_type=jnp.float32)

    return pl.pallas_call(
        body, grid=(k // bk,),
        in_specs=[pl.BlockSpec((m, bk), lambda j: (0, j)),
                  pl.BlockSpec((bk, C), lambda j: (j, 0)),
                  pl.BlockSpec((m, C), lambda j: (0, 0))],
        out_specs=[pl.BlockSpec((m, C), lambda j: (0, 0)),
                   pl.BlockSpec((bk, C), lambda j: (j, 0))],
        out_shape=[jax.ShapeDtypeStruct((m, C), jnp.float32),
                   jax.ShapeDtypeStruct((k, C), jnp.float32)],
        compiler_params=_PARAMS1,
    )(b, xr, xc)


def _rank0_chain(lap, x0, u1, w0):
    m = lap.shape[0]

    def body(l_ref, x0_ref, u1_ref, w_ref, o_ref):
        lb = _bf(l_ref[...])
        r0 = jnp.concatenate([_bf(x0_ref[...]), _bf(u1_ref[...])], axis=1)
        z1 = jnp.dot(lb, r0, preferred_element_type=jnp.float32)
        z2 = jnp.dot(lb, _bf(z1), preferred_element_type=jnp.float32)
        xall = jnp.concatenate([r0, _bf(z1), _bf(z2)], axis=1)
        o_ref[...] = jnp.dot(xall, _bf(w_ref[...]),
                             preferred_element_type=jnp.float32)

    return pl.pallas_call(
        body,
        in_specs=[pl.BlockSpec(lap.shape, lambda: (0, 0)),
                  pl.BlockSpec((m, C), lambda: (0, 0)),
                  pl.BlockSpec((m, C), lambda: (0, 0)),
                  pl.BlockSpec(w0.shape, lambda: (0, 0))],
        out_specs=pl.BlockSpec((m, C), lambda: (0, 0)),
        out_shape=jax.ShapeDtypeStruct((m, C), jnp.float32),
        compiler_params=pltpu.CompilerParams(vmem_limit_bytes=_VMEM),
    )(lap, x0, u1, w0)


def kernel(x_0, x_1, x_2, laplacian_0, laplacian_down_1, laplacian_up_1,
           laplacian_down_2, laplacian_up_2, b1, b2,
           weight_0, weight_1, weight_2):
    u1, l1 = _dual(b1, x_1, x_0, bk=1024)
    u2, l2 = _dual(b2, x_2, x_1, bk=256)

    w0 = weight_0.transpose(2, 0, 1)
    w0 = w0[jnp.array([0, 3, 1, 4, 2, 5])].reshape(6 * C, C)
    y_0 = _rank0_chain(laplacian_0, x_0, u1, w0)

    w1 = weight_1.transpose(2, 0, 1)
    w1 = w1.at[6].add(w1[8]).at[7].add(w1[9])
    zd1a = _lap_pass(laplacian_down_1, [x_1, l1, u2])
    zu1a = _lap_pass(laplacian_up_1, [l1, u2])
    ws1 = w1[jnp.array([0, 5, 10, 6, 1, 11, 3, 13])].reshape(8 * C, C)
    wd1 = w1[jnp.array([7, 2, 12])].reshape(3 * C, C)
    wu1 = w1[jnp.array([4, 14])].reshape(2 * C, C)
    y1p = _lap_pass2_combine(laplacian_down_1, zd1a,
                             [l1, x_1, u2, zd1a, zu1a], wd1, ws1, None)
    y_1 = _lap_pass2_combine(laplacian_up_1, zu1a, [], wu1, None, y1p)

    w2 = weight_2.transpose(2, 0, 1)
    w2 = w2.at[1].add(w2[3]).at[2].add(w2[4])
    wr2 = w2[jnp.array([5, 0])].reshape(2 * C, C)
    wz1d2 = w2[jnp.array([6, 1])].reshape(2 * C, C)
    wz2d2 = w2[jnp.array([7, 2])].reshape(2 * C, C)
    wz1u2 = w2[jnp.array([8])].reshape(C, C)
    wz2u2 = w2[jnp.array([9])].reshape(C, C)
    y2p = _cheb_chain(laplacian_down_2, [x_2, l2], wr2, wz1d2, wz2d2, None)
    y_2 = _cheb_chain(laplacian_up_2, [x_2], None, wz1u2, wz2u2, y2p)

    return (y_0, y_1, y_2)
